# Initial kernel scaffold; baseline (speedup 1.0000x reference)
import jax
import jax.numpy as jnp
from jax.experimental import pallas as pl


def kernel(x, edge_index, batch, W1, b1, W2, b2):
    raise NotImplementedError("write your pallas kernel here")


# speedup vs baseline: 31.1444x; 31.1444x over previous
import functools

import jax
import jax.numpy as jnp
from jax import lax
from jax.experimental import pallas as pl
from jax.experimental.pallas import tpu as pltpu
from jax.experimental.pallas import tpu_sc as plsc

N = 10000
E = 320000
F_IN = 128
HID = 16
C_OUT = 2
G = 64

NC = 2
NS = 16
NW = NC * NS

NP = 10240
EPAD = 327680
EPW = EPAD // NW
CHUNK = 1024
NCHUNK = EPW // CHUNK
RPC = CHUNK // 128
RPT = NP // NS

_MESH = plsc.VectorSubcoreMesh(core_axis_name="c", subcore_axis_name="s")


@functools.partial(
    pl.kernel,
    out_type=jax.ShapeDtypeStruct((NW, NP), jnp.float32),
    mesh=_MESH,
    scratch_types=[
        pltpu.VMEM((CHUNK,), jnp.int32),
        pltpu.VMEM((NP,), jnp.float32),
    ],
    compiler_params=pltpu.CompilerParams(needs_layout_passes=False),
)
def _deg_kernel(dst_hbm, out_hbm, idx_v, deg_v):
    wid = lax.axis_index("s") * NC + lax.axis_index("c")

    def zero_body(i, carry):
        deg_v[pl.ds(i * 16, 16)] = jnp.zeros((16,), jnp.float32)
        return carry

    lax.fori_loop(0, NP // 16, zero_body, 0)

    ones = jnp.ones((16,), jnp.float32)
    base = wid * EPW

    def chunk_body(k, carry):
        pltpu.sync_copy(dst_hbm.at[pl.ds(base + k * CHUNK, CHUNK)], idx_v)
        for t in range(CHUNK // 16):
            v = idx_v[pl.ds(t * 16, 16)]
            plsc.addupdate_scatter(deg_v, [v], ones)
        return carry

    lax.fori_loop(0, NCHUNK, chunk_body, 0)
    pltpu.sync_copy(deg_v, out_hbm.at[wid])


@functools.partial(
    pl.kernel,
    out_type=jax.ShapeDtypeStruct((NC, NP, HID), jnp.float32),
    mesh=_MESH,
    scratch_types=[
        pltpu.VMEM((RPC, 128), jnp.int32),
        pltpu.VMEM((RPC, 128), jnp.int32),
        pltpu.VMEM((CHUNK, HID), jnp.float32),
        pltpu.VMEM((16, HID), jnp.float32),
        pltpu.VMEM_SHARED((NP, HID), jnp.float32),
        pltpu.SemaphoreType.DMA,
    ],
    compiler_params=pltpu.CompilerParams(
        needs_layout_passes=False, use_tc_tiling_on_sc=False),
)
def _scatter_kernel(src_hbm, dst_hbm, tbl_hbm, out_hbm,
                    sidx, didx, rows, zrows, acc, sem):
    cid = lax.axis_index("c")
    sid = lax.axis_index("s")
    wid = sid * NC + cid

    for i in range(16):
        zrows[i, :] = jnp.zeros((HID,), jnp.float32)

    def zero_body(k, carry):
        pltpu.sync_copy(zrows, acc.at[pl.ds(sid * RPT + k * 16, 16)])
        return carry

    lax.fori_loop(0, RPT // 16, zero_body, 0)
    plsc.subcore_barrier()

    rbase = wid * (EPW // 128)

    def chunk_body(k, carry):
        pltpu.sync_copy(src_hbm.at[pl.ds(rbase + k * RPC, RPC)], sidx)
        pltpu.sync_copy(dst_hbm.at[pl.ds(rbase + k * RPC, RPC)], didx)
        descs = [
            pltpu.async_copy(tbl_hbm.at[sidx.at[j]],
                             rows.at[pl.ds(j * 128, 128)], sem)
            for j in range(RPC)
        ]
        for d in descs:
            d.wait()
        for j in range(RPC):
            pltpu.sync_copy(rows.at[pl.ds(j * 128, 128)],
                            acc.at[didx.at[j]], add=True)
        return carry

    lax.fori_loop(0, NCHUNK, chunk_body, 0)
    plsc.subcore_barrier()
    pltpu.sync_copy(acc.at[pl.ds(sid * RPT, RPT)],
                    out_hbm.at[cid, pl.ds(sid * RPT, RPT)])


def _dinv_body(degp_ref, dinv_ref):
    s = jnp.sum(degp_ref[...], axis=0, keepdims=True) + 1.0
    dinv_ref[...] = lax.rsqrt(s)


_dinv_call = pl.pallas_call(
    _dinv_body, out_shape=jax.ShapeDtypeStruct((1, NP), jnp.float32))


def _hprime_body(x_ref, w1_ref, dinv_ref, out_ref):
    mm = jnp.dot(x_ref[...], w1_ref[...], preferred_element_type=jnp.float32)
    out_ref[...] = mm * dinv_ref[...]


_hprime_call = pl.pallas_call(
    _hprime_body, out_shape=jax.ShapeDtypeStruct((NP, HID), jnp.float32))


def _mid_body(p0_ref, p1_ref, hp_ref, dinv_ref, b1_ref, out_ref):
    tot = p0_ref[...] + p1_ref[...] + hp_ref[...]
    pre = tot * dinv_ref[...] + b1_ref[...]
    out_ref[...] = jnp.maximum(pre, 0.0) * dinv_ref[...]


_mid_call = pl.pallas_call(
    _mid_body, out_shape=jax.ShapeDtypeStruct((NP, HID), jnp.float32))


def _final_body(p0_ref, p1_ref, r1p_ref, dinv_ref, batch_ref, w2_ref,
                b2_ref, out_ref):
    t = (p0_ref[...] + p1_ref[...] + r1p_ref[...]) * dinv_ref[...]
    gids = lax.broadcasted_iota(jnp.int32, (G, NP), 0)
    onehot = jnp.where(batch_ref[...] == gids, 1.0, 0.0)
    pooled = jnp.dot(onehot, t, preferred_element_type=jnp.float32)
    cnt = jnp.sum(onehot, axis=1, keepdims=True)
    num = jnp.dot(pooled, w2_ref[...],
                  preferred_element_type=jnp.float32) + cnt * b2_ref[...]
    out_ref[...] = num / jnp.maximum(cnt, 1.0)


_final_call = pl.pallas_call(
    _final_body, out_shape=jax.ShapeDtypeStruct((G, C_OUT), jnp.float32))


def kernel(x, edge_index, batch, W1, b1, W2, b2):
    src = edge_index[0]
    dst = edge_index[1]
    pad = EPAD - E
    srcp = jnp.concatenate([src, jnp.full((pad,), N, jnp.int32)])
    dstp = jnp.concatenate([dst, jnp.full((pad,), N, jnp.int32)])
    src2d = srcp.reshape(EPAD // 128, 128)
    dst2d = dstp.reshape(EPAD // 128, 128)
    xp = jnp.pad(x, ((0, NP - N), (0, 0)))
    bpad = jnp.concatenate(
        [batch, jnp.full((NP - N,), G, jnp.int32)]).reshape(1, NP)

    degp = _deg_kernel(dstp)
    dinv_col = _dinv_call(degp).reshape(NP, 1)
    hp = _hprime_call(xp, W1, dinv_col)
    parts1 = _scatter_kernel(src2d, dst2d, hp)
    r1p = _mid_call(parts1[0], parts1[1], hp, dinv_col, b1.reshape(1, HID))
    parts2 = _scatter_kernel(src2d, dst2d, r1p)
    out = _final_call(parts2[0], parts2[1], r1p, dinv_col, bpad, W2,
                      b2.reshape(1, C_OUT))
    return out


# speedup vs baseline: 33.4036x; 1.0725x over previous
import functools

import jax
import jax.numpy as jnp
from jax import lax
from jax.experimental import pallas as pl
from jax.experimental.pallas import tpu as pltpu
from jax.experimental.pallas import tpu_sc as plsc

N = 10000
E = 320000
F_IN = 128
HID = 16
C_OUT = 2
G = 64

NC = 2
NS = 16
NW = NC * NS

NP = 10240
EPAD = 327680
EPW = EPAD // NW
CHUNK = 1024
NCHUNK = EPW // CHUNK
RPC = CHUNK // 128
RPT = NP // NS

_MESH = plsc.VectorSubcoreMesh(core_axis_name="c", subcore_axis_name="s")


@functools.partial(
    pl.kernel,
    out_type=jax.ShapeDtypeStruct((NW, NP), jnp.float32),
    mesh=_MESH,
    scratch_types=[
        pltpu.VMEM((CHUNK,), jnp.int32),
        pltpu.VMEM((NP,), jnp.float32),
    ],
    compiler_params=pltpu.CompilerParams(needs_layout_passes=False),
)
def _deg_kernel(dst_hbm, out_hbm, idx_v, deg_v):
    wid = lax.axis_index("s") * NC + lax.axis_index("c")

    def zero_body(i, carry):
        deg_v[pl.ds(i * 16, 16)] = jnp.zeros((16,), jnp.float32)
        return carry

    lax.fori_loop(0, NP // 16, zero_body, 0)

    ones = jnp.ones((16,), jnp.float32)
    base = wid * EPW

    def chunk_body(k, carry):
        pltpu.sync_copy(dst_hbm.at[pl.ds(base + k * CHUNK, CHUNK)], idx_v)
        for t in range(CHUNK // 16):
            v = idx_v[pl.ds(t * 16, 16)]
            plsc.addupdate_scatter(deg_v, [v], ones)
        return carry

    lax.fori_loop(0, NCHUNK, chunk_body, 0)
    pltpu.sync_copy(deg_v, out_hbm.at[wid])


@functools.partial(
    pl.kernel,
    out_type=jax.ShapeDtypeStruct((NC, NP, HID), jnp.float32),
    mesh=_MESH,
    scratch_types=[
        pltpu.VMEM((2, RPC, 128), jnp.int32),
        pltpu.VMEM((2, RPC, 128), jnp.int32),
        pltpu.VMEM((2, CHUNK, HID), jnp.float32),
        pltpu.VMEM((16, HID), jnp.float32),
        pltpu.VMEM_SHARED((NP, HID), jnp.float32),
        pltpu.SemaphoreType.DMA,
        pltpu.SemaphoreType.DMA,
        pltpu.SemaphoreType.DMA,
        pltpu.SemaphoreType.DMA,
    ],
    compiler_params=pltpu.CompilerParams(
        needs_layout_passes=False, use_tc_tiling_on_sc=False),
)
def _scatter_kernel(src_hbm, dst_hbm, tbl_hbm, out_hbm,
                    sidx, didx, rows, zrows, acc,
                    sg0, sg1, ss0, ss1):
    cid = lax.axis_index("c")
    sid = lax.axis_index("s")
    wid = sid * NC + cid
    sems_g = (sg0, sg1)
    sems_s = (ss0, ss1)

    for i in range(16):
        zrows[i, :] = jnp.zeros((HID,), jnp.float32)

    def zero_body(k, carry):
        pltpu.sync_copy(zrows, acc.at[pl.ds(sid * RPT + k * 16, 16)])
        return carry

    lax.fori_loop(0, RPT // 16, zero_body, 0)
    plsc.subcore_barrier()

    rbase = wid * (EPW // 128)

    def chunk_body(m, carry):
        for b in range(2):
            k = 2 * m + b

            @pl.when(m > 0)
            def _():
                pltpu.make_async_copy(
                    tbl_hbm.at[pl.ds(0, CHUNK)], rows.at[b],
                    sems_s[b]).wait()

            pltpu.sync_copy(src_hbm.at[pl.ds(rbase + k * RPC, RPC)],
                            sidx.at[b])
            pltpu.sync_copy(dst_hbm.at[pl.ds(rbase + k * RPC, RPC)],
                            didx.at[b])
            gd = [
                pltpu.async_copy(tbl_hbm.at[sidx.at[b, j]],
                                 rows.at[b, pl.ds(j * 128, 128)],
                                 sems_g[b])
                for j in range(RPC)
            ]
            for j in range(RPC):
                gd[j].wait()
                pltpu.async_copy(rows.at[b, pl.ds(j * 128, 128)],
                                 acc.at[didx.at[b, j]], sems_s[b],
                                 add=True)
        return carry

    lax.fori_loop(0, NCHUNK // 2, chunk_body, 0)
    for b in range(2):
        pltpu.make_async_copy(
            tbl_hbm.at[pl.ds(0, CHUNK)], rows.at[b], sems_s[b]).wait()
    plsc.subcore_barrier()
    pltpu.sync_copy(acc.at[pl.ds(sid * RPT, RPT)],
                    out_hbm.at[cid, pl.ds(sid * RPT, RPT)])


def _dinv_body(degp_ref, dinv_ref):
    s = jnp.sum(degp_ref[...], axis=0, keepdims=True) + 1.0
    dinv_ref[...] = lax.rsqrt(s)


_dinv_call = pl.pallas_call(
    _dinv_body, out_shape=jax.ShapeDtypeStruct((1, NP), jnp.float32))


def _hprime_body(x_ref, w1_ref, dinv_ref, out_ref):
    mm = jnp.dot(x_ref[...], w1_ref[...], preferred_element_type=jnp.float32)
    out_ref[...] = mm * dinv_ref[...]


_hprime_call = pl.pallas_call(
    _hprime_body, out_shape=jax.ShapeDtypeStruct((NP, HID), jnp.float32))


def _mid_body(p0_ref, p1_ref, hp_ref, dinv_ref, b1_ref, out_ref):
    tot = p0_ref[...] + p1_ref[...] + hp_ref[...]
    pre = tot * dinv_ref[...] + b1_ref[...]
    out_ref[...] = jnp.maximum(pre, 0.0) * dinv_ref[...]


_mid_call = pl.pallas_call(
    _mid_body, out_shape=jax.ShapeDtypeStruct((NP, HID), jnp.float32))


def _final_body(p0_ref, p1_ref, r1p_ref, dinv_ref, batch_ref, w2_ref,
                b2_ref, out_ref):
    t = (p0_ref[...] + p1_ref[...] + r1p_ref[...]) * dinv_ref[...]
    gids = lax.broadcasted_iota(jnp.int32, (G, NP), 0)
    onehot = jnp.where(batch_ref[...] == gids, 1.0, 0.0)
    pooled = jnp.dot(onehot, t, preferred_element_type=jnp.float32)
    cnt = jnp.sum(onehot, axis=1, keepdims=True)
    num = jnp.dot(pooled, w2_ref[...],
                  preferred_element_type=jnp.float32) + cnt * b2_ref[...]
    out_ref[...] = num / jnp.maximum(cnt, 1.0)


_final_call = pl.pallas_call(
    _final_body, out_shape=jax.ShapeDtypeStruct((G, C_OUT), jnp.float32))


def kernel(x, edge_index, batch, W1, b1, W2, b2):
    src = edge_index[0]
    dst = edge_index[1]
    pad = EPAD - E
    srcp = jnp.concatenate([src, jnp.full((pad,), N, jnp.int32)])
    dstp = jnp.concatenate([dst, jnp.full((pad,), N, jnp.int32)])
    src2d = srcp.reshape(EPAD // 128, 128)
    dst2d = dstp.reshape(EPAD // 128, 128)
    xp = jnp.pad(x, ((0, NP - N), (0, 0)))
    bpad = jnp.concatenate(
        [batch, jnp.full((NP - N,), G, jnp.int32)]).reshape(1, NP)

    degp = _deg_kernel(dstp)
    dinv_col = _dinv_call(degp).reshape(NP, 1)
    hp = _hprime_call(xp, W1, dinv_col)
    parts1 = _scatter_kernel(src2d, dst2d, hp)
    r1p = _mid_call(parts1[0], parts1[1], hp, dinv_col, b1.reshape(1, HID))
    parts2 = _scatter_kernel(src2d, dst2d, r1p)
    out = _final_call(parts2[0], parts2[1], r1p, dinv_col, bpad, W2,
                      b2.reshape(1, C_OUT))
    return out


# speedup vs baseline: 39.4120x; 1.1799x over previous
import functools

import jax
import jax.numpy as jnp
from jax import lax
from jax.experimental import pallas as pl
from jax.experimental.pallas import tpu as pltpu
from jax.experimental.pallas import tpu_sc as plsc

N = 10000
E = 320000
F_IN = 128
HID = 16
C_OUT = 2
G = 64

NC = 2
NS = 16
NW = NC * NS

NP = 10240
EPAD = 327680
CHUNK = 1024
RPC = CHUNK // 128
RPT = NP // NS

EPW_EDGE = (14336, 6144)
EPW_DEG = (12288, 8192)
_EDGE_C1_BASE = 16 * EPW_EDGE[0]
_DEG_C1_BASE = 16 * EPW_DEG[0]

_MESH = plsc.VectorSubcoreMesh(core_axis_name="c", subcore_axis_name="s")


@functools.partial(
    pl.kernel,
    out_type=jax.ShapeDtypeStruct((NW, NP), jnp.float32),
    mesh=_MESH,
    scratch_types=[
        pltpu.VMEM((CHUNK,), jnp.int32),
        pltpu.VMEM((NP,), jnp.float32),
    ],
    compiler_params=pltpu.CompilerParams(needs_layout_passes=False),
)
def _deg_kernel(dst_hbm, out_hbm, idx_v, deg_v):
    cid = lax.axis_index("c")
    sid = lax.axis_index("s")
    wid = sid * NC + cid

    def zero_body(i, carry):
        deg_v[pl.ds(i * 16, 16)] = jnp.zeros((16,), jnp.float32)
        return carry

    lax.fori_loop(0, NP // 16, zero_body, 0)

    ones = jnp.ones((16,), jnp.float32)
    base = jnp.where(cid == 0, sid * EPW_DEG[0],
                     _DEG_C1_BASE + sid * EPW_DEG[1])
    nchunk = jnp.where(cid == 0, EPW_DEG[0] // CHUNK, EPW_DEG[1] // CHUNK)

    def chunk_body(k, carry):
        pltpu.sync_copy(dst_hbm.at[pl.ds(base + k * CHUNK, CHUNK)], idx_v)
        for t in range(CHUNK // 16):
            v = idx_v[pl.ds(t * 16, 16)]
            plsc.addupdate_scatter(deg_v, [v], ones)
        return carry

    lax.fori_loop(0, nchunk, chunk_body, 0)
    pltpu.sync_copy(deg_v, out_hbm.at[wid])


@functools.partial(
    pl.kernel,
    out_type=jax.ShapeDtypeStruct((NC, NP, HID), jnp.float32),
    mesh=_MESH,
    scratch_types=[
        pltpu.VMEM((2, RPC, 128), jnp.int32),
        pltpu.VMEM((2, RPC, 128), jnp.int32),
        pltpu.VMEM((2, CHUNK, HID), jnp.float32),
        pltpu.VMEM((16, HID), jnp.float32),
        pltpu.VMEM_SHARED((NP, HID), jnp.float32),
        pltpu.SemaphoreType.DMA,
        pltpu.SemaphoreType.DMA,
        pltpu.SemaphoreType.DMA,
        pltpu.SemaphoreType.DMA,
    ],
    compiler_params=pltpu.CompilerParams(
        needs_layout_passes=False, use_tc_tiling_on_sc=False),
)
def _scatter_kernel(src_hbm, dst_hbm, tbl_hbm, out_hbm,
                    sidx, didx, rows, zrows, acc,
                    sg0, sg1, ss0, ss1):
    cid = lax.axis_index("c")
    sid = lax.axis_index("s")
    wid = sid * NC + cid
    sems_g = (sg0, sg1)
    sems_s = (ss0, ss1)

    for i in range(16):
        zrows[i, :] = jnp.zeros((HID,), jnp.float32)

    def zero_body(k, carry):
        pltpu.sync_copy(zrows, acc.at[pl.ds(sid * RPT + k * 16, 16)])
        return carry

    lax.fori_loop(0, RPT // 16, zero_body, 0)
    plsc.subcore_barrier()

    rbase = jnp.where(cid == 0, sid * (EPW_EDGE[0] // 128),
                      (_EDGE_C1_BASE // 128) + sid * (EPW_EDGE[1] // 128))
    npair = jnp.where(cid == 0, EPW_EDGE[0] // (2 * CHUNK),
                      EPW_EDGE[1] // (2 * CHUNK))

    def chunk_body(m, carry):
        for b in range(2):
            k = 2 * m + b

            @pl.when(m > 0)
            def _():
                pltpu.make_async_copy(
                    tbl_hbm.at[pl.ds(0, CHUNK)], rows.at[b],
                    sems_s[b]).wait()

            pltpu.sync_copy(src_hbm.at[pl.ds(rbase + k * RPC, RPC)],
                            sidx.at[b])
            pltpu.sync_copy(dst_hbm.at[pl.ds(rbase + k * RPC, RPC)],
                            didx.at[b])
            gd = [
                pltpu.async_copy(tbl_hbm.at[sidx.at[b, j]],
                                 rows.at[b, pl.ds(j * 128, 128)],
                                 sems_g[b])
                for j in range(RPC)
            ]
            for j in range(RPC):
                gd[j].wait()
                pltpu.async_copy(rows.at[b, pl.ds(j * 128, 128)],
                                 acc.at[didx.at[b, j]], sems_s[b],
                                 add=True)
        return carry

    lax.fori_loop(0, npair, chunk_body, 0)
    for b in range(2):
        pltpu.make_async_copy(
            tbl_hbm.at[pl.ds(0, CHUNK)], rows.at[b], sems_s[b]).wait()
    plsc.subcore_barrier()
    pltpu.sync_copy(acc.at[pl.ds(sid * RPT, RPT)],
                    out_hbm.at[cid, pl.ds(sid * RPT, RPT)])


def _hprime_body(x_ref, w1_ref, degp_ref, out_ref, dinv_ref):
    s = jnp.sum(degp_ref[...], axis=0, keepdims=True) + 1.0
    dinv_col = jnp.transpose(lax.rsqrt(s), (1, 0))
    dinv_ref[...] = dinv_col
    mm = jnp.dot(x_ref[...], w1_ref[...], preferred_element_type=jnp.float32)
    out_ref[pl.ds(0, N), :] = mm * dinv_col[0:N, :]
    out_ref[pl.ds(N, NP - N), :] = jnp.zeros((NP - N, HID), jnp.float32)


_hprime_call = pl.pallas_call(
    _hprime_body,
    out_shape=(jax.ShapeDtypeStruct((NP, HID), jnp.float32),
               jax.ShapeDtypeStruct((NP, 1), jnp.float32)))


def _mid_body(p0_ref, p1_ref, hp_ref, dinv_ref, b1_ref, out_ref):
    tot = p0_ref[...] + p1_ref[...] + hp_ref[...]
    pre = tot * dinv_ref[...] + b1_ref[...]
    out_ref[...] = jnp.maximum(pre, 0.0) * dinv_ref[...]


_mid_call = pl.pallas_call(
    _mid_body, out_shape=jax.ShapeDtypeStruct((NP, HID), jnp.float32))


def _final_body(p0_ref, p1_ref, r1p_ref, dinv_ref, batch_ref, w2_ref,
                b2_ref, out_ref):
    t = (p0_ref[...] + p1_ref[...] + r1p_ref[...]) * dinv_ref[...]
    gids = lax.broadcasted_iota(jnp.int32, (G, NP), 0)
    onehot = jnp.where(batch_ref[...] == gids, 1.0, 0.0)
    pooled = jnp.dot(onehot, t, preferred_element_type=jnp.float32)
    cnt = jnp.sum(onehot, axis=1, keepdims=True)
    num = jnp.dot(pooled, w2_ref[...],
                  preferred_element_type=jnp.float32) + cnt * b2_ref[...]
    out_ref[...] = num / jnp.maximum(cnt, 1.0)


_final_call = pl.pallas_call(
    _final_body, out_shape=jax.ShapeDtypeStruct((G, C_OUT), jnp.float32))


def kernel(x, edge_index, batch, W1, b1, W2, b2):
    src = edge_index[0]
    dst = edge_index[1]
    pad = EPAD - E
    srcp = jnp.concatenate([src, jnp.full((pad,), N, jnp.int32)])
    dstp = jnp.concatenate([dst, jnp.full((pad,), N, jnp.int32)])
    src2d = srcp.reshape(EPAD // 128, 128)
    dst2d = dstp.reshape(EPAD // 128, 128)
    bpad = jnp.concatenate(
        [batch, jnp.full((NP - N,), G, jnp.int32)]).reshape(1, NP)

    degp = _deg_kernel(dstp)
    hp, dinv_col = _hprime_call(x, W1, degp)
    parts1 = _scatter_kernel(src2d, dst2d, hp)
    r1p = _mid_call(parts1[0], parts1[1], hp, dinv_col, b1.reshape(1, HID))
    parts2 = _scatter_kernel(src2d, dst2d, r1p)
    out = _final_call(parts2[0], parts2[1], r1p, dinv_col, bpad, W2,
                      b2.reshape(1, C_OUT))
    return out


# speedup vs baseline: 42.4158x; 1.0762x over previous
import functools

import jax
import jax.numpy as jnp
from jax import lax
from jax.experimental import pallas as pl
from jax.experimental.pallas import tpu as pltpu
from jax.experimental.pallas import tpu_sc as plsc

N = 10000
E = 320000
F_IN = 128
HID = 16
C_OUT = 2
G = 64

NC = 2
NS = 16
NW = NC * NS

NP = 10240
EPAD = 327680
CHUNK = 1024
RPC = CHUNK // 128
RPT = NP // NS

EPW_EDGE = (14336, 6144)
EPW_DEG = (12288, 8192)
_EDGE_C1_BASE = 16 * EPW_EDGE[0]
_DEG_C1_BASE = 16 * EPW_DEG[0]

_MESH = plsc.VectorSubcoreMesh(core_axis_name="c", subcore_axis_name="s")


@functools.partial(
    pl.kernel,
    out_type=jax.ShapeDtypeStruct((NW, NP), jnp.float32),
    mesh=_MESH,
    scratch_types=[
        pltpu.VMEM((CHUNK,), jnp.int32),
        pltpu.VMEM((NP,), jnp.float32),
    ],
    compiler_params=pltpu.CompilerParams(needs_layout_passes=False),
)
def _deg_kernel(dst_hbm, out_hbm, idx_v, deg_v):
    cid = lax.axis_index("c")
    sid = lax.axis_index("s")
    wid = sid * NC + cid

    def zero_body(i, carry):
        deg_v[pl.ds(i * 16, 16)] = jnp.zeros((16,), jnp.float32)
        return carry

    lax.fori_loop(0, NP // 16, zero_body, 0)

    ones = jnp.ones((16,), jnp.float32)
    base = jnp.where(cid == 0, sid * EPW_DEG[0],
                     _DEG_C1_BASE + sid * EPW_DEG[1])
    nchunk = jnp.where(cid == 0, EPW_DEG[0] // CHUNK, EPW_DEG[1] // CHUNK)

    def chunk_body(k, carry):
        pltpu.sync_copy(dst_hbm.at[pl.ds(base + k * CHUNK, CHUNK)], idx_v)
        for t in range(CHUNK // 16):
            v = idx_v[pl.ds(t * 16, 16)]
            plsc.addupdate_scatter(deg_v, [v], ones)
        return carry

    lax.fori_loop(0, nchunk, chunk_body, 0)
    pltpu.sync_copy(deg_v, out_hbm.at[wid])


@functools.partial(
    pl.kernel,
    out_type=jax.ShapeDtypeStruct((NC, NP, HID), jnp.float32),
    mesh=_MESH,
    scratch_types=[
        pltpu.VMEM((2, RPC, 2, 128), jnp.int32),
        pltpu.VMEM((2, CHUNK, HID), jnp.float32),
        pltpu.VMEM_SHARED((NP, HID), jnp.float32),
        pltpu.SemaphoreType.DMA,
        pltpu.SemaphoreType.DMA,
        pltpu.SemaphoreType.DMA,
        pltpu.SemaphoreType.DMA,
    ],
    compiler_params=pltpu.CompilerParams(
        needs_layout_passes=False, use_tc_tiling_on_sc=False),
)
def _scatter_kernel(ei_hbm, tbl_hbm, out_hbm,
                    eidx, rows, acc,
                    sg0, sg1, ss0, ss1):
    cid = lax.axis_index("c")
    sid = lax.axis_index("s")
    sems_g = (sg0, sg1)
    sems_s = (ss0, ss1)

    pltpu.sync_copy(tbl_hbm.at[pl.ds(sid * RPT, RPT)],
                    acc.at[pl.ds(sid * RPT, RPT)])
    plsc.subcore_barrier()

    rbase = jnp.where(cid == 0, sid * (EPW_EDGE[0] // 128),
                      (_EDGE_C1_BASE // 128) + sid * (EPW_EDGE[1] // 128))
    npair = jnp.where(cid == 0, EPW_EDGE[0] // (2 * CHUNK),
                      EPW_EDGE[1] // (2 * CHUNK))

    def chunk_body(m, carry):
        for b in range(2):
            k = 2 * m + b

            @pl.when(m > 0)
            def _():
                pltpu.make_async_copy(
                    tbl_hbm.at[pl.ds(0, CHUNK)], rows.at[b],
                    sems_s[b]).wait()

            pltpu.sync_copy(ei_hbm.at[pl.ds(rbase + k * RPC, RPC)],
                            eidx.at[b])
            gd = [
                pltpu.async_copy(tbl_hbm.at[eidx.at[b, j, 0]],
                                 rows.at[b, pl.ds(j * 128, 128)],
                                 sems_g[b])
                for j in range(RPC)
            ]
            for j in range(RPC):
                gd[j].wait()
                pltpu.async_copy(rows.at[b, pl.ds(j * 128, 128)],
                                 acc.at[eidx.at[b, j, 1]], sems_s[b],
                                 add=True)
        return carry

    lax.fori_loop(0, npair, chunk_body, 0)
    for b in range(2):
        pltpu.make_async_copy(
            tbl_hbm.at[pl.ds(0, CHUNK)], rows.at[b], sems_s[b]).wait()
    plsc.subcore_barrier()
    pltpu.sync_copy(acc.at[pl.ds(sid * RPT, RPT)],
                    out_hbm.at[cid, pl.ds(sid * RPT, RPT)])


def _hprime_body(x_ref, w1_ref, degp_ref, out_ref, dinv_ref):
    s = jnp.sum(degp_ref[...], axis=0, keepdims=True) + 1.0
    dinv_col = jnp.transpose(lax.rsqrt(s), (1, 0))
    dinv_ref[...] = dinv_col
    mm = jnp.dot(x_ref[...], w1_ref[...], preferred_element_type=jnp.float32)
    out_ref[pl.ds(0, N), :] = mm * dinv_col[0:N, :]
    out_ref[pl.ds(N, NP - N), :] = jnp.zeros((NP - N, HID), jnp.float32)


_hprime_call = pl.pallas_call(
    _hprime_body,
    out_shape=(jax.ShapeDtypeStruct((NP, HID), jnp.float32),
               jax.ShapeDtypeStruct((NP, 1), jnp.float32)))


def _mid_body(p0_ref, p1_ref, hp_ref, dinv_ref, b1_ref, out_ref):
    tot = p0_ref[...] + p1_ref[...] - hp_ref[...]
    pre = tot * dinv_ref[...] + b1_ref[...]
    out_ref[...] = jnp.maximum(pre, 0.0) * dinv_ref[...]


_mid_call = pl.pallas_call(
    _mid_body, out_shape=jax.ShapeDtypeStruct((NP, HID), jnp.float32))


def _final_body(p0_ref, p1_ref, r1p_ref, dinv_ref, batch_ref, w2_ref,
                b2_ref, out_ref):
    t = (p0_ref[...] + p1_ref[...] - r1p_ref[...]) * dinv_ref[...]
    gids = lax.broadcasted_iota(jnp.int32, (G, NP), 0)
    onehot = jnp.where(batch_ref[...] == gids, 1.0, 0.0)
    pooled = jnp.dot(onehot, t, preferred_element_type=jnp.float32)
    cnt = jnp.sum(onehot, axis=1, keepdims=True)
    num = jnp.dot(pooled, w2_ref[...],
                  preferred_element_type=jnp.float32) + cnt * b2_ref[...]
    out_ref[...] = num / jnp.maximum(cnt, 1.0)


_final_call = pl.pallas_call(
    _final_body, out_shape=jax.ShapeDtypeStruct((G, C_OUT), jnp.float32))


def kernel(x, edge_index, batch, W1, b1, W2, b2):
    pad = EPAD - E
    ep = jnp.concatenate(
        [edge_index, jnp.full((2, pad), N, jnp.int32)], axis=1)
    ei_pair = jnp.stack(
        [ep[0].reshape(EPAD // 128, 128), ep[1].reshape(EPAD // 128, 128)],
        axis=1)
    bpad = jnp.concatenate(
        [batch, jnp.full((NP - N,), G, jnp.int32)]).reshape(1, NP)

    degp = _deg_kernel(ep[1])
    hp, dinv_col = _hprime_call(x, W1, degp)
    parts1 = _scatter_kernel(ei_pair, hp)
    r1p = _mid_call(parts1[0], parts1[1], hp, dinv_col, b1.reshape(1, HID))
    parts2 = _scatter_kernel(ei_pair, r1p)
    out = _final_call(parts2[0], parts2[1], r1p, dinv_col, bpad, W2,
                      b2.reshape(1, C_OUT))
    return out


# speedup vs baseline: 48.5116x; 1.1437x over previous
import functools

import jax
import jax.numpy as jnp
from jax import lax
from jax.experimental import pallas as pl
from jax.experimental.pallas import tpu as pltpu
from jax.experimental.pallas import tpu_sc as plsc

N = 10000
E = 320000
F_IN = 128
HID = 16
C_OUT = 2
G = 64

NC = 2
NS = 16
NW = NC * NS

NP = 10240
EPAD = 327680
CHUNK = 1024
RPC = CHUNK // 128
RPT = NP // NS

EPW_EDGE = (16384, 4096)
EPW_DEG = (13312, 7168)
_EDGE_C1_BASE = 16 * EPW_EDGE[0]
_DEG_C1_BASE = 16 * EPW_DEG[0]

_MESH = plsc.VectorSubcoreMesh(core_axis_name="c", subcore_axis_name="s")


@functools.partial(
    pl.kernel,
    out_type=jax.ShapeDtypeStruct((NW, NP), jnp.float32),
    mesh=_MESH,
    scratch_types=[
        pltpu.VMEM((CHUNK,), jnp.int32),
        pltpu.VMEM((NP,), jnp.float32),
    ],
    compiler_params=pltpu.CompilerParams(needs_layout_passes=False),
)
def _deg_kernel(dst_hbm, out_hbm, idx_v, deg_v):
    cid = lax.axis_index("c")
    sid = lax.axis_index("s")
    wid = sid * NC + cid

    def zero_body(i, carry):
        deg_v[pl.ds(i * 16, 16)] = jnp.zeros((16,), jnp.float32)
        return carry

    lax.fori_loop(0, NP // 16, zero_body, 0)

    ones = jnp.ones((16,), jnp.float32)
    base = jnp.where(cid == 0, sid * EPW_DEG[0],
                     _DEG_C1_BASE + sid * EPW_DEG[1])
    nchunk = jnp.where(cid == 0, EPW_DEG[0] // CHUNK, EPW_DEG[1] // CHUNK)

    def chunk_body(k, carry):
        pltpu.sync_copy(dst_hbm.at[pl.ds(base + k * CHUNK, CHUNK)], idx_v)
        for t in range(CHUNK // 16):
            v = idx_v[pl.ds(t * 16, 16)]
            plsc.addupdate_scatter(deg_v, [v], ones)
        return carry

    lax.fori_loop(0, nchunk, chunk_body, 0)
    pltpu.sync_copy(deg_v, out_hbm.at[wid])


@functools.partial(
    pl.kernel,
    out_type=jax.ShapeDtypeStruct((NC, NP, HID), jnp.float32),
    mesh=_MESH,
    scratch_types=[
        pltpu.VMEM((2, RPC, 2, 128), jnp.int32),
        pltpu.VMEM((2, CHUNK, HID), jnp.float32),
        pltpu.VMEM_SHARED((NP, HID), jnp.float32),
        pltpu.SemaphoreType.DMA,
        pltpu.SemaphoreType.DMA,
        pltpu.SemaphoreType.DMA,
        pltpu.SemaphoreType.DMA,
    ],
    compiler_params=pltpu.CompilerParams(
        needs_layout_passes=False, use_tc_tiling_on_sc=False),
)
def _scatter_kernel(ei_hbm, tbl_hbm, zero_hbm, out_hbm,
                    eidx, rows, acc,
                    sg0, sg1, ss0, ss1):
    cid = lax.axis_index("c")
    sid = lax.axis_index("s")
    sems_g = (sg0, sg1)
    sems_s = (ss0, ss1)

    @pl.when(cid == 0)
    def _():
        pltpu.sync_copy(tbl_hbm.at[pl.ds(sid * RPT, RPT)],
                        acc.at[pl.ds(sid * RPT, RPT)])

    @pl.when(cid != 0)
    def _():
        pltpu.sync_copy(zero_hbm.at[pl.ds(sid * RPT, RPT)],
                        acc.at[pl.ds(sid * RPT, RPT)])

    plsc.subcore_barrier()

    rbase = jnp.where(cid == 0, sid * (EPW_EDGE[0] // 128),
                      (_EDGE_C1_BASE // 128) + sid * (EPW_EDGE[1] // 128))
    npair = jnp.where(cid == 0, EPW_EDGE[0] // (2 * CHUNK),
                      EPW_EDGE[1] // (2 * CHUNK))

    def chunk_body(m, carry):
        for b in range(2):
            k = 2 * m + b

            @pl.when(m > 0)
            def _():
                pltpu.make_async_copy(
                    tbl_hbm.at[pl.ds(0, CHUNK)], rows.at[b],
                    sems_s[b]).wait()

            pltpu.sync_copy(ei_hbm.at[pl.ds(rbase + k * RPC, RPC)],
                            eidx.at[b])
            gd = [
                pltpu.async_copy(tbl_hbm.at[eidx.at[b, j, 0]],
                                 rows.at[b, pl.ds(j * 128, 128)],
                                 sems_g[b])
                for j in range(RPC)
            ]
            for j in range(RPC):
                gd[j].wait()
                pltpu.async_copy(rows.at[b, pl.ds(j * 128, 128)],
                                 acc.at[eidx.at[b, j, 1]], sems_s[b],
                                 add=True)
        return carry

    lax.fori_loop(0, npair, chunk_body, 0)
    for b in range(2):
        pltpu.make_async_copy(
            tbl_hbm.at[pl.ds(0, CHUNK)], rows.at[b], sems_s[b]).wait()
    plsc.subcore_barrier()
    pltpu.sync_copy(acc.at[pl.ds(sid * RPT, RPT)],
                    out_hbm.at[cid, pl.ds(sid * RPT, RPT)])


PR = NP * HID // 128


def _hprime_body(x_ref, w1_ref, degp_ref, out_ref, dinv_ref):
    s = jnp.sum(degp_ref[...], axis=0, keepdims=True) + 1.0
    dinv_col = jnp.transpose(lax.rsqrt(s), (1, 0))
    dinv_ref[...] = dinv_col
    mm = jnp.dot(x_ref[...], w1_ref[...], preferred_element_type=jnp.float32)
    out_ref[pl.ds(0, N), :] = mm * dinv_col[0:N, :]
    out_ref[pl.ds(N, NP - N), :] = jnp.zeros((NP - N, HID), jnp.float32)


_hprime_call = pl.pallas_call(
    _hprime_body,
    out_shape=(jax.ShapeDtypeStruct((NP, HID), jnp.float32),
               jax.ShapeDtypeStruct((NP, 1), jnp.float32)))


def _mid_body(p0_ref, p1_ref, dinvp_ref, b1_ref, out_ref):
    pre = (p0_ref[...] + p1_ref[...]) * dinvp_ref[...] + b1_ref[...]
    out_ref[...] = jnp.maximum(pre, 0.0) * dinvp_ref[...]


_mid_call = pl.pallas_call(
    _mid_body, out_shape=jax.ShapeDtypeStruct((PR, 128), jnp.float32))


def _final_body(p0_ref, p1_ref, dinv_ref, batch_ref, w2_ref,
                b2_ref, out_ref):
    t = (p0_ref[...] + p1_ref[...]) * dinv_ref[...]
    gids = lax.broadcasted_iota(jnp.int32, (G, NP), 0)
    onehot = jnp.where(batch_ref[...] == gids, 1.0, 0.0)
    pooled = jnp.dot(onehot, t, preferred_element_type=jnp.float32)
    cnt = jnp.sum(onehot, axis=1, keepdims=True)
    num = jnp.dot(pooled, w2_ref[...],
                  preferred_element_type=jnp.float32) + cnt * b2_ref[...]
    out_ref[...] = num / jnp.maximum(cnt, 1.0)


_final_call = pl.pallas_call(
    _final_body, out_shape=jax.ShapeDtypeStruct((G, C_OUT), jnp.float32))


def kernel(x, edge_index, batch, W1, b1, W2, b2):
    pad = EPAD - E
    ep = jnp.concatenate(
        [edge_index, jnp.full((2, pad), N, jnp.int32)], axis=1)
    ei_pair = jnp.stack(
        [ep[0].reshape(EPAD // 128, 128), ep[1].reshape(EPAD // 128, 128)],
        axis=1)
    bpad = jnp.concatenate(
        [batch, jnp.full((NP - N,), G, jnp.int32)]).reshape(1, NP)

    z16 = jnp.zeros((NP, HID), jnp.float32)
    degp = _deg_kernel(ep[1])
    hp, dinv_col = _hprime_call(x, W1, degp)
    dinvp = jnp.broadcast_to(
        dinv_col.reshape(NP, 1), (NP, HID)).reshape(PR, 128)
    b1t = jnp.tile(b1.reshape(1, HID), (1, 8))
    parts1 = _scatter_kernel(ei_pair, hp, z16)
    pp1 = parts1.reshape(NC, PR, 128)
    r1p_p = _mid_call(pp1[0], pp1[1], dinvp, b1t)
    parts2 = _scatter_kernel(ei_pair, r1p_p.reshape(NP, HID), z16)
    out = _final_call(parts2[0], parts2[1], dinv_col, bpad, W2,
                      b2.reshape(1, C_OUT))
    return out
